# baseline (speedup 1.0000x reference)
import jax
import jax.numpy as jnp
from jax import lax
from jax.experimental import pallas as pl
from jax.experimental.pallas import tpu as pltpu

N = 32768
H1 = 16
H2 = 32
E = 8
CHUNK = 4096
GRID = N // CHUNK


def _body(x_ref, w1_ref, b1_ref, w2_ref, b2_ref, w3_ref, b3_ref, out_ref):
    x = x_ref[...]
    h1 = jnp.maximum(w1_ref[...] * x + b1_ref[...], 0.0)
    h2 = lax.dot_general(w2_ref[...], h1, (((0,), (0,)), ((), ())),
                         preferred_element_type=jnp.float32)
    h2 = jnp.maximum(h2 + b2_ref[...], 0.0)
    lg = lax.dot_general(w3_ref[...], h2, (((0,), (0,)), ((), ())),
                         preferred_element_type=jnp.float32)
    lg = lg + b3_ref[...]
    m = jnp.max(lg, axis=0, keepdims=True)
    p = jnp.exp(lg - m)
    s = jnp.sum(p, axis=0, keepdims=True)
    out_ref[...] = p / s


def kernel(snr_estimate, temperature, W1, b1, W2, b2, W3, b3):
    inv_t = 1.0 / temperature
    outT = pl.pallas_call(
        _body,
        grid=(GRID,),
        in_specs=[
            pl.BlockSpec((1, CHUNK), lambda i: (0, i)),
            pl.BlockSpec((H1, 1), lambda i: (0, 0)),
            pl.BlockSpec((H1, 1), lambda i: (0, 0)),
            pl.BlockSpec((H1, H2), lambda i: (0, 0)),
            pl.BlockSpec((H2, 1), lambda i: (0, 0)),
            pl.BlockSpec((H2, E), lambda i: (0, 0)),
            pl.BlockSpec((E, 1), lambda i: (0, 0)),
        ],
        out_specs=pl.BlockSpec((E, CHUNK), lambda i: (0, i)),
        out_shape=jax.ShapeDtypeStruct((E, N), jnp.float32),
        compiler_params=pltpu.CompilerParams(
            dimension_semantics=("arbitrary",),
        ),
    )(
        snr_estimate.reshape(1, N),
        W1.reshape(H1, 1), b1.reshape(H1, 1),
        W2, b2.reshape(H2, 1),
        W3 * inv_t, (b3 * inv_t).reshape(E, 1),
    )
    return outT.T


# speedup vs baseline: 1.3616x; 1.3616x over previous
import jax
import jax.numpy as jnp
from jax import lax
from jax.experimental import pallas as pl
from jax.experimental.pallas import tpu as pltpu

N = 32768
H1 = 16
H2 = 32
E = 8


def _body(x_ref, w1_ref, b1_ref, w2_ref, b2_ref, w3_ref, b3_ref, out_ref):
    x = x_ref[...]
    h1 = jnp.maximum(w1_ref[...] * x + b1_ref[...], 0.0)
    h2 = lax.dot_general(w2_ref[...], h1, (((0,), (0,)), ((), ())),
                         preferred_element_type=jnp.float32)
    h2 = jnp.maximum(h2 + b2_ref[...], 0.0)
    lg = lax.dot_general(w3_ref[...], h2, (((0,), (0,)), ((), ())),
                         preferred_element_type=jnp.float32)
    lg = lg + b3_ref[...]
    m = jnp.max(lg, axis=0, keepdims=True)
    p = jnp.exp(lg - m)
    s = jnp.sum(p, axis=0, keepdims=True)
    out_ref[...] = p / s


def kernel(snr_estimate, temperature, W1, b1, W2, b2, W3, b3):
    inv_t = 1.0 / temperature
    outT = pl.pallas_call(
        _body,
        out_shape=jax.ShapeDtypeStruct((E, N), jnp.float32),
    )(
        snr_estimate.reshape(1, N),
        W1.reshape(H1, 1), b1.reshape(H1, 1),
        W2, b2.reshape(H2, 1),
        W3 * inv_t, (b3 * inv_t).reshape(E, 1),
    )
    return outT.T
